# Initial kernel scaffold; baseline (speedup 1.0000x reference)
import jax
import jax.numpy as jnp
from jax.experimental import pallas as pl


def kernel(x, edge_index, pos_edge_index, neg_edge_index, W1_0, b1_0, W2_0, b2_0, W1_1, b1_1, W2_1, b2_1, W1_2, b1_2, W2_2, b2_2, Wp1, bp1, Wp2, bp2):
    raise NotImplementedError("write your pallas kernel here")


# speedup vs baseline: 2.0728x; 2.0728x over previous
import functools

import jax
import jax.numpy as jnp
from jax import lax
from jax.experimental import pallas as pl
from jax.experimental.pallas import tpu as pltpu
from jax.experimental.pallas import tpu_sc as plsc

N = 10000
D = 128
NC = 2
NS = 16
L = 16
NW = NC * NS
CH = 128
NPAD = 10240
ZROWS = NPAD // NS


def _mesh():
    return plsc.VectorSubcoreMesh(
        core_axis_name="c", subcore_axis_name="s", num_cores=NC, num_subcores=NS
    )


def _pad_edges(idx, fill):
    e = idx.shape[0]
    nchunk = -(-e // (NW * CH))
    nchunk = -(-nchunk // 8) * 8
    ep = nchunk * NW * CH
    if ep != e:
        idx = jnp.concatenate([idx, jnp.full((ep - e,), fill, jnp.int32)])
    return idx.reshape(NW * nchunk, CH), nchunk


@functools.cache
def _make_agg(nchunk):
    @functools.partial(
        pl.kernel,
        out_type=jax.ShapeDtypeStruct((NC, NPAD, D), jnp.float32),
        mesh=_mesh(),
        compiler_params=pltpu.CompilerParams(needs_layout_passes=False),
        scratch_types=[
            pltpu.VMEM((nchunk, CH), jnp.int32),
            pltpu.VMEM((nchunk, CH), jnp.int32),
            pltpu.VMEM((CH, D), jnp.float32),
            pltpu.VMEM_SHARED((NPAD, D), jnp.float32),
            pltpu.SemaphoreType.DMA,
        ],
    )
    def agg(z_hbm, row_hbm, col_hbm, out_hbm, ridx, cidx, gbuf, acc, sem):
        c = lax.axis_index("c")
        s = lax.axis_index("s")
        w = c * NS + s

        zero = jnp.zeros((L,), jnp.float32)

        def zrow(i, carry):
            for k in range(D // L):
                gbuf[i, pl.ds(k * L, L)] = zero
            return carry

        lax.fori_loop(0, CH, zrow, 0)
        for t in range(ZROWS // CH):
            pltpu.sync_copy(gbuf, acc.at[pl.ds(s * ZROWS + t * CH, CH)])
        plsc.subcore_barrier()

        pltpu.sync_copy(row_hbm.at[pl.ds(w * nchunk, nchunk)], ridx)
        pltpu.sync_copy(col_hbm.at[pl.ds(w * nchunk, nchunk)], cidx)

        def body(j, carry):
            pltpu.async_copy(z_hbm.at[ridx.at[j]], gbuf, sem).wait()
            pltpu.sync_copy(gbuf, acc.at[cidx.at[j]], add=True)
            return carry

        lax.fori_loop(0, nchunk, body, 0)
        plsc.subcore_barrier()

        for t in range(ZROWS // CH):
            sl = pl.ds(s * ZROWS + t * CH, CH)
            pltpu.sync_copy(acc.at[sl], out_hbm.at[c, sl])

    return agg


def _mlp_body(relu_out, z_ref, a0_ref, a1_ref, w1_ref, b1_ref, w2_ref, b2_ref,
              o_ref):
    h = z_ref[...] + a0_ref[...] + a1_ref[...]
    h1 = jnp.maximum(
        jnp.dot(h, w1_ref[...], preferred_element_type=jnp.float32)
        + b1_ref[...], 0.0)
    h2 = jnp.dot(h1, w2_ref[...], preferred_element_type=jnp.float32) + b2_ref[...]
    o_ref[...] = jnp.maximum(h2, 0.0) if relu_out else h2


_ROWS_BLK = 1000


@functools.cache
def _make_mlp(relu_out):
    vec = pl.BlockSpec((_ROWS_BLK, D), lambda i: (i, 0))
    mat = pl.BlockSpec((D, D), lambda i: (0, 0))
    bias = pl.BlockSpec((1, D), lambda i: (0, 0))
    return pl.pallas_call(
        functools.partial(_mlp_body, relu_out),
        grid=(N // _ROWS_BLK,),
        in_specs=[vec, vec, vec, mat, bias, mat, bias],
        out_specs=vec,
        out_shape=jax.ShapeDtypeStruct((N, D), jnp.float32),
    )


def _ab_body(z_ref, wa_ref, wb_ref, bp1_ref, a_ref, b_ref):
    z = z_ref[...]
    a_ref[...] = (
        jnp.dot(z, wa_ref[...], preferred_element_type=jnp.float32) + bp1_ref[...]
    )
    b_ref[...] = jnp.dot(z, wb_ref[...], preferred_element_type=jnp.float32)


@functools.cache
def _make_ab():
    vec = pl.BlockSpec((_ROWS_BLK, D), lambda i: (i, 0))
    mat = pl.BlockSpec((D, D), lambda i: (0, 0))
    bias = pl.BlockSpec((1, D), lambda i: (0, 0))
    return pl.pallas_call(
        _ab_body,
        grid=(N // _ROWS_BLK,),
        in_specs=[vec, mat, mat, bias],
        out_specs=(vec, vec),
        out_shape=(
            jax.ShapeDtypeStruct((N, D), jnp.float32),
            jax.ShapeDtypeStruct((N, D), jnp.float32),
        ),
    )


@functools.cache
def _make_pred(nchunk):
    @functools.partial(
        pl.kernel,
        out_type=jax.ShapeDtypeStruct((NW * nchunk, CH), jnp.float32),
        mesh=_mesh(),
        compiler_params=pltpu.CompilerParams(needs_layout_passes=False),
        scratch_types=[
            pltpu.VMEM((nchunk, CH), jnp.int32),
            pltpu.VMEM((nchunk, CH), jnp.int32),
            pltpu.VMEM((CH, D), jnp.float32),
            pltpu.VMEM((CH, D), jnp.float32),
            pltpu.VMEM((nchunk, CH), jnp.float32),
            pltpu.VMEM((D,), jnp.float32),
            pltpu.VMEM((L,), jnp.float32),
            pltpu.VMEM((L, L), jnp.float32),
            pltpu.SemaphoreType.DMA,
            pltpu.SemaphoreType.DMA,
        ],
    )
    def pred(a_hbm, b_hbm, src_hbm, dst_hbm, w_hbm, bias_hbm, out_hbm,
             sidx, didx, ga, gb, obuf, wv, bv, tbuf, sem_a, sem_b):
        c = lax.axis_index("c")
        s = lax.axis_index("s")
        w = c * NS + s

        pltpu.sync_copy(w_hbm, wv)
        pltpu.sync_copy(bias_hbm, bv)
        pltpu.sync_copy(src_hbm.at[pl.ds(w * nchunk, nchunk)], sidx)
        pltpu.sync_copy(dst_hbm.at[pl.ds(w * nchunk, nchunk)], didx)

        bvec = bv[...]
        ws = [wv[pl.ds(k * L, L)] for k in range(D // L)]
        lanes = lax.iota(jnp.int32, L)

        def chunk(j, carry):
            cpa = pltpu.async_copy(a_hbm.at[sidx.at[j]], ga, sem_a)
            cpb = pltpu.async_copy(b_hbm.at[didx.at[j]], gb, sem_b)
            cpa.wait()
            cpb.wait()
            for g in range(CH // L):
                for lane in range(L):
                    e = g * L + lane
                    acc = bvec
                    for k in range(D // L):
                        va = ga[e, pl.ds(k * L, L)]
                        vb = gb[e, pl.ds(k * L, L)]
                        acc = acc + jnp.maximum(va + vb, 0.0) * ws[k]
                    tbuf[lane, :] = acc
                ov = jnp.zeros((L,), jnp.float32)
                for d2 in range(L):
                    col = jnp.full((L,), d2, jnp.int32)
                    ov = ov + plsc.load_gather(tbuf, [lanes, col])
                obuf[j, pl.ds(g * L, L)] = ov
            return carry

        lax.fori_loop(0, nchunk, chunk, 0)
        pltpu.sync_copy(obuf, out_hbm.at[pl.ds(w * nchunk, nchunk)])

    return pred


def kernel(x, edge_index, pos_edge_index, neg_edge_index,
           W1_0, b1_0, W2_0, b2_0,
           W1_1, b1_1, W2_1, b2_1,
           W1_2, b1_2, W2_2, b2_2,
           Wp1, bp1, Wp2, bp2):
    gin = [(W1_0, b1_0, W2_0, b2_0, True),
           (W1_1, b1_1, W2_1, b2_1, True),
           (W1_2, b1_2, W2_2, b2_2, False)]

    row2d, nchunk_e = _pad_edges(edge_index[0], 0)
    col2d, _ = _pad_edges(edge_index[1], N)
    agg_fn = _make_agg(nchunk_e)

    z = x
    for W1, b1, W2, b2, relu_out in gin:
        parts = agg_fn(z, row2d, col2d)
        z = _make_mlp(relu_out)(
            z, parts[0, :N], parts[1, :N],
            W1, b1.reshape(1, D), W2, b2.reshape(1, D))

    a_nodes, b_nodes = _make_ab()(
        z, Wp1[:D], Wp1[D:], bp1.reshape(1, D))

    wp2 = Wp2.reshape(D)
    bp2v = jnp.full((L,), bp2[0] / L, jnp.float32)

    def predict(ei):
        e = ei.shape[1]
        src2d, nchunk = _pad_edges(ei[0], 0)
        dst2d, _ = _pad_edges(ei[1], 0)
        out = _make_pred(nchunk)(a_nodes, b_nodes, src2d, dst2d, wp2, bp2v)
        return out.reshape(-1)[:e]

    return predict(pos_edge_index), predict(neg_edge_index)


# speedup vs baseline: 2.5765x; 1.2430x over previous
import functools

import jax
import jax.numpy as jnp
from jax import lax
from jax.experimental import pallas as pl
from jax.experimental.pallas import tpu as pltpu
from jax.experimental.pallas import tpu_sc as plsc

N = 10000
D = 128
NC = 2
NS = 16
L = 16
NW = NC * NS
CH = 128
GI = 16
NPAD = 10240
ZROWS = NPAD // NS


def _mesh():
    return plsc.VectorSubcoreMesh(
        core_axis_name="c", subcore_axis_name="s", num_cores=NC, num_subcores=NS
    )


def _pad_edges(idx, fill):
    e = idx.shape[0]
    nchunk = -(-e // (NW * CH))
    nchunk = -(-nchunk // GI) * GI
    ep = nchunk * NW * CH
    if ep != e:
        idx = jnp.concatenate([idx, jnp.full((ep - e,), fill, jnp.int32)])
    return idx.reshape(NW * nchunk, CH), nchunk


@functools.cache
def _make_agg(nchunk):
    @functools.partial(
        pl.kernel,
        out_type=jax.ShapeDtypeStruct((NC, NPAD, D), jnp.float32),
        mesh=_mesh(),
        compiler_params=pltpu.CompilerParams(needs_layout_passes=False),
        scratch_types=[
            pltpu.VMEM((GI, CH), jnp.int32),
            pltpu.VMEM((GI, CH), jnp.int32),
            pltpu.VMEM((nchunk, CH), jnp.int32),
            pltpu.VMEM((CH, D), jnp.float32),
            pltpu.VMEM((CH, D), jnp.float32),
            pltpu.VMEM_SHARED((NPAD, D), jnp.float32),
            pltpu.SemaphoreType.DMA,
            pltpu.SemaphoreType.DMA,
            pltpu.SemaphoreType.DMA,
            pltpu.SemaphoreType.DMA,
        ],
    )
    def agg(z_hbm, row_hbm, col_hbm, out_hbm, ri0, ri1,
            cidx, gb0, gb1, acc, sr0, sr1, sg0, sg1):
        c = lax.axis_index("c")
        s = lax.axis_index("s")
        w = c * NS + s
        ribufs, rsems = [ri0, ri1], [sr0, sr1]
        gbufs, gsems = [gb0, gb1], [sg0, sg1]
        ngrp = nchunk // GI

        zero = jnp.zeros((L,), jnp.float32)

        def zrow(i, carry):
            for k in range(D // L):
                gb0[i, pl.ds(k * L, L)] = zero
            return carry

        lax.fori_loop(0, CH, zrow, 0)
        for t in range(ZROWS // CH):
            pltpu.sync_copy(gb0, acc.at[pl.ds(s * ZROWS + t * CH, CH)])
        plsc.subcore_barrier()

        pltpu.sync_copy(col_hbm.at[pl.ds(w * nchunk, nchunk)], cidx)

        def ridx_start(g):
            pltpu.async_copy(
                row_hbm.at[pl.ds(w * nchunk + g * GI, GI)],
                ribufs[g % 2], rsems[g % 2])

        ridx_start(0)

        for g in range(ngrp):
            rb = ribufs[g % 2]
            pltpu.make_async_copy(
                row_hbm.at[pl.ds(w * nchunk, GI)], rb, rsems[g % 2]).wait()
            if g + 1 < ngrp:
                ridx_start(g + 1)
            pltpu.async_copy(z_hbm.at[rb.at[0]], gbufs[0], gsems[0])

            def pair(tt, carry):
                for b in range(2):
                    t = tt * 2 + b
                    pltpu.make_async_copy(
                        z_hbm.at[rb.at[0]], gbufs[b], gsems[b]).wait()

                    @pl.when(t + 1 < GI)
                    def _():
                        pltpu.async_copy(
                            z_hbm.at[rb.at[t + 1]], gbufs[1 - b],
                            gsems[1 - b])

                    pltpu.sync_copy(
                        gbufs[b], acc.at[cidx.at[g * GI + t]], add=True)
                return carry

            lax.fori_loop(0, GI // 2, pair, 0)
        plsc.subcore_barrier()

        for t in range(ZROWS // CH):
            sl = pl.ds(s * ZROWS + t * CH, CH)
            pltpu.sync_copy(acc.at[sl], out_hbm.at[c, sl])

    return agg


def _bf(t):
    return t.astype(jnp.bfloat16)


def _mlp_body(relu_out, z_ref, a0_ref, a1_ref, w1_ref, b1_ref, w2_ref, b2_ref,
              o_ref):
    h = z_ref[...] + a0_ref[...] + a1_ref[...]
    h1 = jnp.maximum(
        jnp.dot(_bf(h), _bf(w1_ref[...]), preferred_element_type=jnp.float32)
        + b1_ref[...], 0.0)
    h2 = (jnp.dot(_bf(h1), _bf(w2_ref[...]), preferred_element_type=jnp.float32)
          + b2_ref[...])
    o_ref[...] = jnp.maximum(h2, 0.0) if relu_out else h2


_ROWS_BLK = 1000


@functools.cache
def _make_mlp(relu_out):
    vec = pl.BlockSpec((_ROWS_BLK, D), lambda i: (i, 0))
    mat = pl.BlockSpec((D, D), lambda i: (0, 0))
    bias = pl.BlockSpec((1, D), lambda i: (0, 0))
    return pl.pallas_call(
        functools.partial(_mlp_body, relu_out),
        grid=(N // _ROWS_BLK,),
        in_specs=[vec, vec, vec, mat, bias, mat, bias],
        out_specs=vec,
        out_shape=jax.ShapeDtypeStruct((N, D), jnp.float32),
    )


def _ab_body(z_ref, wa_ref, wb_ref, bp1_ref, a_ref, b_ref):
    z = _bf(z_ref[...])
    a_ref[...] = (
        jnp.dot(z, _bf(wa_ref[...]), preferred_element_type=jnp.float32)
        + bp1_ref[...])
    b_ref[...] = jnp.dot(z, _bf(wb_ref[...]), preferred_element_type=jnp.float32)


@functools.cache
def _make_ab():
    vec = pl.BlockSpec((_ROWS_BLK, D), lambda i: (i, 0))
    mat = pl.BlockSpec((D, D), lambda i: (0, 0))
    bias = pl.BlockSpec((1, D), lambda i: (0, 0))
    return pl.pallas_call(
        _ab_body,
        grid=(N // _ROWS_BLK,),
        in_specs=[vec, mat, mat, bias],
        out_specs=(vec, vec),
        out_shape=(
            jax.ShapeDtypeStruct((N, D), jnp.float32),
            jax.ShapeDtypeStruct((N, D), jnp.float32),
        ),
    )


@functools.cache
def _make_pred(nchunk):
    @functools.partial(
        pl.kernel,
        out_type=jax.ShapeDtypeStruct((NW * nchunk, CH), jnp.float32),
        mesh=_mesh(),
        compiler_params=pltpu.CompilerParams(needs_layout_passes=False),
        scratch_types=[
            pltpu.VMEM((nchunk, CH), jnp.int32),
            pltpu.VMEM((nchunk, CH), jnp.int32),
            pltpu.VMEM((CH, D), jnp.float32),
            pltpu.VMEM((CH, D), jnp.float32),
            pltpu.VMEM((CH, D), jnp.float32),
            pltpu.VMEM((CH, D), jnp.float32),
            pltpu.VMEM((nchunk, CH), jnp.float32),
            pltpu.VMEM((D,), jnp.float32),
            pltpu.VMEM((L,), jnp.float32),
            pltpu.SemaphoreType.DMA,
            pltpu.SemaphoreType.DMA,
            pltpu.SemaphoreType.DMA,
            pltpu.SemaphoreType.DMA,
        ],
    )
    def pred(a_hbm, b_hbm, src_hbm, dst_hbm, w_hbm, bias_hbm, out_hbm,
             sidx, didx, ga0, ga1, gb0, gb1, obuf, wv, bv,
             sa0, sa1, sb0, sb1):
        c = lax.axis_index("c")
        s = lax.axis_index("s")
        w = c * NS + s
        gas, gbs = [ga0, ga1], [gb0, gb1]
        sas, sbs = [sa0, sa1], [sb0, sb1]

        pltpu.sync_copy(w_hbm, wv)
        pltpu.sync_copy(bias_hbm, bv)
        pltpu.sync_copy(src_hbm.at[pl.ds(w * nchunk, nchunk)], sidx)
        pltpu.sync_copy(dst_hbm.at[pl.ds(w * nchunk, nchunk)], didx)

        bvec = bv[...]
        ws = [wv[pl.ds(k * L, L)] for k in range(D // L)]
        lanes = lax.iota(jnp.int32, L)

        def start(j, b):
            pltpu.async_copy(a_hbm.at[sidx.at[j]], gas[b], sas[b])
            pltpu.async_copy(b_hbm.at[didx.at[j]], gbs[b], sbs[b])

        start(0, 0)

        def bfr(v):
            u = plsc.bitcast(v, jnp.uint32)
            u = (u + jnp.uint32(0x7FFF)
                 + ((u >> jnp.uint32(16)) & jnp.uint32(1)))
            u = u & jnp.uint32(0xFFFF0000)
            return plsc.bitcast(u, jnp.float32)

        rots = [((lanes + sh) % L).astype(jnp.int32) for sh in (8, 4, 2, 1)]

        def hsum(v):
            for perm in rots:
                v = v + v.at[perm].get(mode="promise_in_bounds")
            return v

        def do_chunk(j, ga, gb):
            for g in range(CH // L):
                ov = jnp.zeros((L,), jnp.float32)
                for lane in range(L):
                    e = g * L + lane
                    acc = bvec
                    for k in range(D // L):
                        va = ga[e, pl.ds(k * L, L)]
                        vb = gb[e, pl.ds(k * L, L)]
                        acc = acc + bfr(jnp.maximum(va + vb, 0.0)) * ws[k]
                    ov = jnp.where(lanes == lane, hsum(acc), ov)
                obuf[j, pl.ds(g * L, L)] = ov

        def body(jj, carry):
            for b in range(2):
                j = jj * 2 + b
                pltpu.make_async_copy(
                    a_hbm.at[sidx.at[j]], gas[b], sas[b]).wait()
                pltpu.make_async_copy(
                    b_hbm.at[didx.at[j]], gbs[b], sbs[b]).wait()

                @pl.when(j + 1 < nchunk)
                def _():
                    start(j + 1, 1 - b)

                do_chunk(j, gas[b], gbs[b])
            return carry

        lax.fori_loop(0, nchunk // 2, body, 0)
        pltpu.sync_copy(obuf, out_hbm.at[pl.ds(w * nchunk, nchunk)])

    return pred


def kernel(x, edge_index, pos_edge_index, neg_edge_index,
           W1_0, b1_0, W2_0, b2_0,
           W1_1, b1_1, W2_1, b2_1,
           W1_2, b1_2, W2_2, b2_2,
           Wp1, bp1, Wp2, bp2):
    gin = [(W1_0, b1_0, W2_0, b2_0, True),
           (W1_1, b1_1, W2_1, b2_1, True),
           (W1_2, b1_2, W2_2, b2_2, False)]

    row2d, nchunk_e = _pad_edges(edge_index[0], 0)
    col2d, _ = _pad_edges(edge_index[1], N)
    agg_fn = _make_agg(nchunk_e)

    z = x
    for W1, b1, W2, b2, relu_out in gin:
        parts = agg_fn(z, row2d, col2d)
        z = _make_mlp(relu_out)(
            z, parts[0, :N], parts[1, :N],
            W1, b1.reshape(1, D), W2, b2.reshape(1, D))

    a_nodes, b_nodes = _make_ab()(
        z, Wp1[:D], Wp1[D:], bp1.reshape(1, D))

    wp2 = Wp2.reshape(D).astype(jnp.bfloat16).astype(jnp.float32)
    bp2v = jnp.full((L,), bp2[0] / L, jnp.float32)

    def predict(ei):
        e = ei.shape[1]
        src2d, nchunk = _pad_edges(ei[0], 0)
        dst2d, _ = _pad_edges(ei[1], 0)
        out = _make_pred(nchunk)(a_nodes, b_nodes, src2d, dst2d, wp2, bp2v)
        return out.reshape(-1)[:e]

    return predict(pos_edge_index), predict(neg_edge_index)
